# baseline (speedup 1.0000x reference)
import functools

import jax
import jax.numpy as jnp
from jax import lax
from jax.experimental import pallas as pl
from jax.experimental.pallas import tpu as pltpu
from jax.experimental.pallas import tpu_sc as plsc

N = 10000
D = 128
E = 320000

NC = 2
NS = 16
NW = NC * NS
CH = 64
K = 160
EPAD = NW * K * CH
NPAD = 10240
SLAB = NPAD // NS
NBUF = 4


def _sc_agg(table, rowp, colp, width):
    mesh = plsc.VectorSubcoreMesh(core_axis_name="c", subcore_axis_name="s")

    @functools.partial(
        pl.kernel,
        out_type=jax.ShapeDtypeStruct((NC, NPAD, width), jnp.float32),
        mesh=mesh,
        scratch_types=[
            pltpu.VMEM((K // 4, CH), jnp.int32),
            pltpu.VMEM((K // 4, CH), jnp.int32),
        ] + [pltpu.VMEM((CH, width), jnp.float32) for _ in range(NBUF)] + [
            pltpu.VMEM_SHARED((NPAD, width), jnp.float32),
        ] + [pltpu.SemaphoreType.DMA for _ in range(2 * NBUF)],
        compiler_params=pltpu.CompilerParams(use_tc_tiling_on_sc=False),
    )
    def agg(table_hbm, rowp_hbm, colp_hbm, out_hbm,
            ridx, cidx, *rest):
        rb = rest[:NBUF]
        acc = rest[NBUF]
        gs = rest[NBUF + 1:2 * NBUF + 1]
        ss = rest[2 * NBUF + 1:]
        cid = lax.axis_index("c")
        sid = lax.axis_index("s")
        wid = cid * NS + sid
        r0 = sid * SLAB
        pltpu.sync_copy(table_hbm.at[pl.ds(r0, SLAB)], acc.at[pl.ds(r0, SLAB)])
        plsc.subcore_barrier()

        HS = K // 4
        for h in range(4):
            pltpu.sync_copy(rowp_hbm.at[wid, pl.ds(h * HS, HS)], ridx)
            pltpu.sync_copy(colp_hbm.at[wid, pl.ds(h * HS, HS)], cidx)
            pltpu.async_copy(table_hbm.at[ridx.at[0]], rb[0], gs[0])
            pltpu.async_copy(table_hbm.at[ridx.at[1]], rb[1], gs[1])

            def step(j, carry):
                for u in range(NBUF):
                    c = NBUF * j + u
                    u2 = (u + 2) % NBUF
                    pltpu.make_async_copy(
                        table_hbm.at[ridx.at[c]], rb[u], gs[u]).wait()
                    pltpu.async_copy(rb[u], acc.at[cidx.at[c]], ss[u], add=True)

                    @pl.when(c >= 2)
                    def _():
                        pltpu.make_async_copy(
                            rb[u2], acc.at[cidx.at[c - 2]], ss[u2]).wait()

                    @pl.when(c + 2 < HS)
                    def _():
                        pltpu.async_copy(
                            table_hbm.at[ridx.at[c + 2]], rb[u2], gs[u2])
                return carry

            lax.fori_loop(0, HS // NBUF, step, 0)
            pltpu.make_async_copy(
                rb[NBUF - 2], acc.at[cidx.at[HS - 2]], ss[NBUF - 2]).wait()
            pltpu.make_async_copy(
                rb[NBUF - 1], acc.at[cidx.at[HS - 1]], ss[NBUF - 1]).wait()
        plsc.subcore_barrier()
        pltpu.sync_copy(acc.at[pl.ds(r0, SLAB)],
                        out_hbm.at[cid, pl.ds(r0, SLAB)])

    return agg(table, rowp, colp)


WD = 16


def _sc_deg(colp):
    mesh = plsc.VectorSubcoreMesh(core_axis_name="c", subcore_axis_name="s")

    @functools.partial(
        pl.kernel,
        out_type=jax.ShapeDtypeStruct((NC, NPAD, WD), jnp.float32),
        mesh=mesh,
        scratch_types=[
            pltpu.VMEM((K, CH), jnp.int32),
            pltpu.VMEM((CH, WD), jnp.float32),
            pltpu.VMEM_SHARED((NPAD, WD), jnp.float32),
        ],
    )
    def deg(colp_hbm, out_hbm, cidx, ones_v, acc):
        cid = lax.axis_index("c")
        sid = lax.axis_index("s")
        wid = cid * NS + sid
        r0 = sid * SLAB

        def fill(i, carry):
            ones_v[i, :] = jnp.ones((WD,), jnp.float32)
            return carry

        lax.fori_loop(0, CH, fill, 0)
        for t in range(SLAB // CH):
            pltpu.sync_copy(ones_v, acc.at[pl.ds(r0 + t * CH, CH)])
        pltpu.sync_copy(colp_hbm.at[wid], cidx)
        plsc.subcore_barrier()

        def step(i, carry):
            pltpu.sync_copy(ones_v, acc.at[cidx.at[i]], add=True)
            return carry

        lax.fori_loop(0, K, step, 0)
        plsc.subcore_barrier()
        pltpu.sync_copy(acc.at[pl.ds(r0, SLAB)],
                        out_hbm.at[cid, pl.ds(r0, SLAB)])

    return deg(colp)


R = 1280
_GRID = NPAD // R


def _dinv_block(dw):
    deg = dw[0, :, 0:1] + dw[1, :, 0:1] - 1.0
    return lax.rsqrt(deg)


def _tc1_body(x_ref, w_ref, dw_ref, o_ref):
    dinv = _dinv_block(dw_ref[...])
    h = jnp.dot(x_ref[...], w_ref[...], preferred_element_type=jnp.float32)
    o_ref[...] = h * dinv


def _tc_mid_body(g_ref, p_ref, dw_ref, b_ref, w_ref, o_ref):
    dinv = _dinv_block(dw_ref[...])
    p = p_ref[...]
    h = jnp.maximum(dinv * (p[0] + p[1] - g_ref[...]) + b_ref[...], 0.0)
    o_ref[...] = jnp.dot(h, w_ref[...], preferred_element_type=jnp.float32) * dinv


def _tc_fin_body(g_ref, p_ref, dw_ref, b_ref, o_ref):
    dinv = _dinv_block(dw_ref[...])
    p = p_ref[...]
    o_ref[...] = dinv * (p[0] + p[1] - g_ref[...]) + b_ref[...]


_spec_rd = pl.BlockSpec((R, D), lambda i: (i, 0))
_spec_w = pl.BlockSpec((D, D), lambda i: (0, 0))
_spec_dw = pl.BlockSpec((NC, R, WD), lambda i: (0, i, 0))
_spec_p = pl.BlockSpec((NC, R, D), lambda i: (0, i, 0))
_spec_b = pl.BlockSpec((1, D), lambda i: (0, 0))

_tc1 = pl.pallas_call(
    _tc1_body, grid=(_GRID,),
    in_specs=[_spec_rd, _spec_w, _spec_dw],
    out_specs=_spec_rd,
    out_shape=jax.ShapeDtypeStruct((NPAD, D), jnp.float32),
)
_tc_mid = pl.pallas_call(
    _tc_mid_body, grid=(_GRID,),
    in_specs=[_spec_rd, _spec_p, _spec_dw, _spec_b, _spec_w],
    out_specs=_spec_rd,
    out_shape=jax.ShapeDtypeStruct((NPAD, D), jnp.float32),
)
_tc_fin = pl.pallas_call(
    _tc_fin_body, grid=(_GRID,),
    in_specs=[_spec_rd, _spec_p, _spec_dw, _spec_b],
    out_specs=_spec_rd,
    out_shape=jax.ShapeDtypeStruct((NPAD, D), jnp.float32),
)


def kernel(x, edge_index, W1, b1, W2, b2):
    pad = EPAD - E
    rowp = jnp.concatenate(
        [edge_index[0], jnp.zeros((pad,), jnp.int32)]).reshape(NW, K, CH)
    colp = jnp.concatenate(
        [edge_index[1], jnp.full((pad,), N, jnp.int32)]).reshape(NW, K, CH)
    x_pad = jnp.zeros((NPAD, D), jnp.float32).at[:N].set(x)
    b1r = b1.reshape(1, D)
    b2r = b2.reshape(1, D)

    dw = _sc_deg(colp)
    g1 = _tc1(x_pad, W1, dw)
    p1 = _sc_agg(g1, rowp, colp, D)
    g2 = _tc_mid(g1, p1, dw, b1r, W2)
    p2 = _sc_agg(g2, rowp, colp, D)
    out = _tc_fin(g2, p2, dw, b2r)
    return out[:N]


# speedup vs baseline: 1.1036x; 1.1036x over previous
import functools

import jax
import jax.numpy as jnp
from jax import lax
from jax.experimental import pallas as pl
from jax.experimental.pallas import tpu as pltpu
from jax.experimental.pallas import tpu_sc as plsc

N = 10000
D = 128
E = 320000

NC = 2
NS = 16
NW = NC * NS
CH = 64
K = 160
EPAD = NW * K * CH
NPAD = 10240
SLAB = NPAD // NS
NBUF = 4


def _sc_agg(table, rowp, colp, width):
    mesh = plsc.VectorSubcoreMesh(core_axis_name="c", subcore_axis_name="s")

    @functools.partial(
        pl.kernel,
        out_type=jax.ShapeDtypeStruct((NC, NPAD, width), jnp.float32),
        mesh=mesh,
        scratch_types=[
            pltpu.VMEM((K // 4, CH), jnp.int32),
            pltpu.VMEM((K // 4, CH), jnp.int32),
        ] + [pltpu.VMEM((CH, width), jnp.float32) for _ in range(NBUF)] + [
            pltpu.VMEM_SHARED((NPAD, width), jnp.float32),
        ] + [pltpu.SemaphoreType.DMA for _ in range(2 * NBUF)],
    )
    def agg(table_hbm, rowp_hbm, colp_hbm, out_hbm,
            ridx, cidx, *rest):
        rb = rest[:NBUF]
        acc = rest[NBUF]
        gs = rest[NBUF + 1:2 * NBUF + 1]
        ss = rest[2 * NBUF + 1:]
        cid = lax.axis_index("c")
        sid = lax.axis_index("s")
        wid = cid * NS + sid
        r0 = sid * SLAB
        pltpu.sync_copy(table_hbm.at[pl.ds(r0, SLAB)], acc.at[pl.ds(r0, SLAB)])
        plsc.subcore_barrier()

        HS = K // 4
        for h in range(4):
            pltpu.sync_copy(rowp_hbm.at[wid, pl.ds(h * HS, HS)], ridx)
            pltpu.sync_copy(colp_hbm.at[wid, pl.ds(h * HS, HS)], cidx)
            pltpu.async_copy(table_hbm.at[ridx.at[0]], rb[0], gs[0])
            pltpu.async_copy(table_hbm.at[ridx.at[1]], rb[1], gs[1])
            pltpu.async_copy(table_hbm.at[ridx.at[2]], rb[2], gs[2])

            def step(j, carry):
                for u in range(NBUF):
                    c = NBUF * j + u
                    u3 = (u + 3) % NBUF
                    pltpu.make_async_copy(
                        table_hbm.at[ridx.at[c]], rb[u], gs[u]).wait()
                    pltpu.async_copy(rb[u], acc.at[cidx.at[c]], ss[u], add=True)

                    @pl.when(c >= 1)
                    def _():
                        pltpu.make_async_copy(
                            rb[u3], acc.at[cidx.at[c - 1]], ss[u3]).wait()

                    @pl.when(c + 3 < HS)
                    def _():
                        pltpu.async_copy(
                            table_hbm.at[ridx.at[c + 3]], rb[u3], gs[u3])
                return carry

            lax.fori_loop(0, HS // NBUF, step, 0)
            pltpu.make_async_copy(
                rb[NBUF - 1], acc.at[cidx.at[HS - 1]], ss[NBUF - 1]).wait()
        plsc.subcore_barrier()
        pltpu.sync_copy(acc.at[pl.ds(r0, SLAB)],
                        out_hbm.at[cid, pl.ds(r0, SLAB)])

    return agg(table, rowp, colp)


WD = 16


def _sc_deg(colp):
    mesh = plsc.VectorSubcoreMesh(core_axis_name="c", subcore_axis_name="s")

    @functools.partial(
        pl.kernel,
        out_type=jax.ShapeDtypeStruct((NC, NPAD, WD), jnp.float32),
        mesh=mesh,
        scratch_types=[
            pltpu.VMEM((K, CH), jnp.int32),
            pltpu.VMEM((CH, WD), jnp.float32),
            pltpu.VMEM_SHARED((NPAD, WD), jnp.float32),
        ],
    )
    def deg(colp_hbm, out_hbm, cidx, ones_v, acc):
        cid = lax.axis_index("c")
        sid = lax.axis_index("s")
        wid = cid * NS + sid
        r0 = sid * SLAB

        def fill(i, carry):
            ones_v[i, :] = jnp.ones((WD,), jnp.float32)
            return carry

        lax.fori_loop(0, CH, fill, 0)
        for t in range(SLAB // CH):
            pltpu.sync_copy(ones_v, acc.at[pl.ds(r0 + t * CH, CH)])
        pltpu.sync_copy(colp_hbm.at[wid], cidx)
        plsc.subcore_barrier()

        def step(i, carry):
            pltpu.sync_copy(ones_v, acc.at[cidx.at[i]], add=True)
            return carry

        lax.fori_loop(0, K, step, 0)
        plsc.subcore_barrier()
        pltpu.sync_copy(acc.at[pl.ds(r0, SLAB)],
                        out_hbm.at[cid, pl.ds(r0, SLAB)])

    return deg(colp)


R = 1280
_GRID = NPAD // R


def _dinv_block(dw):
    deg = dw[0, :, 0:1] + dw[1, :, 0:1] - 1.0
    return lax.rsqrt(deg)


def _tc1_body(x_ref, w_ref, dw_ref, o_ref):
    dinv = _dinv_block(dw_ref[...])
    h = jnp.dot(x_ref[...], w_ref[...], preferred_element_type=jnp.float32)
    o_ref[...] = h * dinv


def _tc_mid_body(g_ref, p_ref, dw_ref, b_ref, w_ref, o_ref):
    dinv = _dinv_block(dw_ref[...])
    p = p_ref[...]
    h = jnp.maximum(dinv * (p[0] + p[1] - g_ref[...]) + b_ref[...], 0.0)
    o_ref[...] = jnp.dot(h, w_ref[...], preferred_element_type=jnp.float32) * dinv


def _tc_fin_body(g_ref, p_ref, dw_ref, b_ref, o_ref):
    dinv = _dinv_block(dw_ref[...])
    p = p_ref[...]
    o_ref[...] = dinv * (p[0] + p[1] - g_ref[...]) + b_ref[...]


_spec_rd = pl.BlockSpec((R, D), lambda i: (i, 0))
_spec_w = pl.BlockSpec((D, D), lambda i: (0, 0))
_spec_dw = pl.BlockSpec((NC, R, WD), lambda i: (0, i, 0))
_spec_p = pl.BlockSpec((NC, R, D), lambda i: (0, i, 0))
_spec_b = pl.BlockSpec((1, D), lambda i: (0, 0))

_tc1 = pl.pallas_call(
    _tc1_body, grid=(_GRID,),
    in_specs=[_spec_rd, _spec_w, _spec_dw],
    out_specs=_spec_rd,
    out_shape=jax.ShapeDtypeStruct((NPAD, D), jnp.float32),
)
_tc_mid = pl.pallas_call(
    _tc_mid_body, grid=(_GRID,),
    in_specs=[_spec_rd, _spec_p, _spec_dw, _spec_b, _spec_w],
    out_specs=_spec_rd,
    out_shape=jax.ShapeDtypeStruct((NPAD, D), jnp.float32),
)
_tc_fin = pl.pallas_call(
    _tc_fin_body, grid=(_GRID,),
    in_specs=[_spec_rd, _spec_p, _spec_dw, _spec_b],
    out_specs=_spec_rd,
    out_shape=jax.ShapeDtypeStruct((NPAD, D), jnp.float32),
)


def kernel(x, edge_index, W1, b1, W2, b2):
    pad = EPAD - E
    rowp = jnp.concatenate(
        [edge_index[0], jnp.zeros((pad,), jnp.int32)]).reshape(NW, K, CH)
    colp = jnp.concatenate(
        [edge_index[1], jnp.full((pad,), N, jnp.int32)]).reshape(NW, K, CH)
    x_pad = jnp.zeros((NPAD, D), jnp.float32).at[:N].set(x)
    b1r = b1.reshape(1, D)
    b2r = b2.reshape(1, D)

    dw = _sc_deg(colp)
    g1 = _tc1(x_pad, W1, dw)
    p1 = _sc_agg(g1, rowp, colp, D)
    g2 = _tc_mid(g1, p1, dw, b1r, W2)
    p2 = _sc_agg(g2, rowp, colp, D)
    out = _tc_fin(g2, p2, dw, b2r)
    return out[:N]
